# baseline (speedup 1.0000x reference)
import jax
import jax.numpy as jnp
from jax.experimental import pallas as pl
from jax.experimental.pallas import tpu as pltpu

N = 4096
D_IN = 64
D_OUT = 64
K = 2
BN = 256
NB = N // BN


def _body(x_ref, adj_ref, w_ref, out_ref, h_ref):
    i = pl.program_id(0)
    k = pl.program_id(1)

    @pl.when((i == 0) & (k == 0))
    def _():
        h_ref[...] = jnp.dot(x_ref[...], w_ref[...],
                             preferred_element_type=jnp.float32)

    part = jnp.dot(adj_ref[...], h_ref[...],
                   preferred_element_type=jnp.float32)

    @pl.when(k == 0)
    def _():
        out_ref[...] = part

    @pl.when(k == K - 1)
    def _():
        out_ref[...] = jnp.maximum(out_ref[...] + part, 0.0)


@jax.jit
def kernel(input, adj_list, W):
    adj_flat = adj_list.reshape(K * N, N)
    return pl.pallas_call(
        _body,
        grid=(NB, K),
        in_specs=[
            pl.BlockSpec((N, D_IN), lambda i, k: (0, 0)),
            pl.BlockSpec((BN, N), lambda i, k: (k * NB + i, 0)),
            pl.BlockSpec((D_IN, D_OUT), lambda i, k: (0, 0)),
        ],
        out_specs=pl.BlockSpec((BN, D_OUT), lambda i, k: (i, 0)),
        out_shape=jax.ShapeDtypeStruct((N, D_OUT), jnp.float32),
        scratch_shapes=[pltpu.VMEM((N, D_OUT), jnp.float32)],
    )(input, adj_flat, W)


# speedup vs baseline: 1.1283x; 1.1283x over previous
import jax
import jax.numpy as jnp
from jax import lax
from jax.experimental import pallas as pl
from jax.experimental.pallas import tpu as pltpu

N = 4096
D_IN = 64
D_OUT = 64
K = 2
BN = 128
NBUF = 8
NB = N // BN
S = K * NB


def _body(x_ref, adj_ref, w_ref, out_ref, ring_ref, sem, h_ref):
    h_ref[...] = jnp.dot(x_ref[...], w_ref[...],
                         preferred_element_type=jnp.float32)

    def dma(s, slot):
        i = s // K
        k = lax.rem(s, K)
        row0 = k * N + i * BN
        return pltpu.make_async_copy(
            adj_ref.at[pl.ds(row0, BN)], ring_ref.at[slot], sem.at[slot])

    for s in range(NBUF):
        dma(s, s).start()

    def step(s, _):
        slot = lax.rem(s, NBUF)
        dma(s, slot).wait()
        part = jnp.dot(ring_ref[slot], h_ref[...],
                       preferred_element_type=jnp.float32)
        i = s // K
        k = lax.rem(s, K)
        rows = pl.ds(i * BN, BN)

        @pl.when(k == 0)
        def _():
            out_ref[rows, :] = part

        @pl.when(k == K - 1)
        def _():
            out_ref[rows, :] = jnp.maximum(out_ref[rows, :] + part, 0.0)

        @pl.when(s + NBUF < S)
        def _():
            dma(s + NBUF, slot).start()

        return 0

    lax.fori_loop(0, S, step, 0)


@jax.jit
def kernel(input, adj_list, W):
    adj_flat = adj_list.reshape(K * N, N)
    return pl.pallas_call(
        _body,
        in_specs=[
            pl.BlockSpec(memory_space=pltpu.VMEM),
            pl.BlockSpec(memory_space=pl.ANY),
            pl.BlockSpec(memory_space=pltpu.VMEM),
        ],
        out_specs=pl.BlockSpec(memory_space=pltpu.VMEM),
        out_shape=jax.ShapeDtypeStruct((N, D_OUT), jnp.float32),
        scratch_shapes=[
            pltpu.VMEM((NBUF, BN, N), jnp.float32),
            pltpu.SemaphoreType.DMA((NBUF,)),
            pltpu.VMEM((N, D_OUT), jnp.float32),
        ],
    )(input, adj_flat, W)


# speedup vs baseline: 1.1988x; 1.0625x over previous
import jax
import jax.numpy as jnp
from jax.experimental import pallas as pl
from jax.experimental.pallas import tpu as pltpu

N = 4096
D_IN = 64
D_OUT = 64
K = 2
BN = 256


def _body(x_ref, adj_ref, w_ref, out_ref, h_ref):
    @pl.when(pl.program_id(0) == 0)
    def _():
        h_ref[...] = jnp.dot(x_ref[...], w_ref[...],
                             preferred_element_type=jnp.float32).astype(
                                 jnp.bfloat16)

    a = (adj_ref[0] + adj_ref[1]).astype(jnp.bfloat16)
    acc = jnp.dot(a, h_ref[...], preferred_element_type=jnp.float32)
    out_ref[...] = jnp.maximum(acc, 0.0)


@jax.jit
def kernel(input, adj_list, W):
    return pl.pallas_call(
        _body,
        grid=(N // BN,),
        in_specs=[
            pl.BlockSpec((N, D_IN), lambda i: (0, 0)),
            pl.BlockSpec((K, BN, N), lambda i: (0, i, 0)),
            pl.BlockSpec((D_IN, D_OUT), lambda i: (0, 0)),
        ],
        out_specs=pl.BlockSpec((BN, D_OUT), lambda i: (i, 0)),
        out_shape=jax.ShapeDtypeStruct((N, D_OUT), jnp.float32),
        scratch_shapes=[pltpu.VMEM((N, D_OUT), jnp.bfloat16)],
    )(input, adj_list, W)
